# Initial kernel scaffold; baseline (speedup 1.0000x reference)
import jax
import jax.numpy as jnp
from jax.experimental import pallas as pl


def kernel(embedding, source_node, context_nodes, neg_samples):
    raise NotImplementedError("write your pallas kernel here")


# speedup vs baseline: 1.5031x; 1.5031x over previous
import functools

import numpy as np
import jax
import jax.numpy as jnp
from jax import lax
from jax.experimental import pallas as pl
from jax.experimental.pallas import tpu as pltpu
from jax.experimental.pallas import tpu_sc as plsc

_VOCAB = 100000
_D = 128
_NW = 16
_PER_W = 25
_ROWS = 32

_MASKS = np.zeros((_NW, 2, _ROWS), np.float32)
for _w in range(_NW):
    for _l in range(1, _PER_W + 1):
        _p = _w * _PER_W + (_l - 1)
        _MASKS[_w, 0 if _p < 200 else 1, _l] = 1.0


def _stable_sigmoid_neg(d):
    t = jnp.exp(-jnp.abs(d))
    num = jnp.where(d >= 0.0, t, jnp.ones_like(t))
    return num / (1.0 + t)


def _sum16(v, lanes):
    for sh in (8, 4, 2, 1):
        perm = jnp.bitwise_xor(lanes, sh)
        v = v + v.at[perm].get(mode="promise_in_bounds")
    return v


def _n2v_body(table_h, idx_h, masks_h, out_h,
              idx_v, rows_v, masks_v, part_v, fin_v, shared, out_v, sem):
    cid = lax.axis_index("c")
    sid = lax.axis_index("s")

    @pl.when(cid == 0)
    def _core0():
        pltpu.sync_copy(idx_h.at[sid], idx_v)
        pltpu.sync_copy(masks_h.at[sid], masks_v)
        pltpu.async_copy(table_h.at[idx_v], rows_v, sem).wait()

        lanes = lax.iota(jnp.int32, 16)
        s_chunks = [rows_v[0, pl.ds(16 * j, 16)] for j in range(8)]
        zacc = jnp.zeros((16,), jnp.float32)
        nacc = jnp.zeros((16,), jnp.float32)
        for g in range(2):
            dots = jnp.zeros((16,), jnp.float32)
            for k in range(16):
                r = g * 16 + k
                acc = rows_v[r, pl.ds(0, 16)] * s_chunks[0]
                for j in range(1, 8):
                    acc = acc + rows_v[r, pl.ds(16 * j, 16)] * s_chunks[j]
                dots = jnp.where(lanes == k, _sum16(acc, lanes), dots)
            mc = masks_v[0, pl.ds(16 * g, 16)]
            mn = masks_v[1, pl.ds(16 * g, 16)]
            zacc = zacc + mc * dots
            nacc = nacc + mn * _stable_sigmoid_neg(dots)

        part_v[0, :] = zacc
        part_v[1, :] = nacc
        pltpu.sync_copy(part_v, shared.at[sid])
        plsc.subcore_barrier()

        @pl.when(sid == 0)
        def _finish():
            pltpu.sync_copy(shared, fin_v)
            zv = fin_v[0, 0, :]
            nv = fin_v[0, 1, :]
            for w in range(1, _NW):
                zv = zv + fin_v[w, 0, :]
                nv = nv + fin_v[w, 1, :]
            zb = _sum16(zv, lanes)
            nb = _sum16(nv, lanes)

            t = jnp.exp(-jnp.abs(zb))
            num = jnp.where(zb >= 0.0, jnp.ones_like(t), t)
            p = num / (1.0 + t)
            p = jnp.clip(p, 1e-7, 1.0 - 1e-7)

            pi = lax.bitcast_convert_type(p, jnp.int32)
            y = (pi.astype(jnp.float32) - 1064866805.0) * 8.262958405176314e-08
            for _ in range(3):
                y = y + p * jnp.exp(-y) - 1.0

            nclip = jnp.clip(nb, 1e-7, 1.0 - 1e-7)
            out_v[...] = -y - nclip
            pltpu.sync_copy(out_v, out_h)


_n2v = functools.partial(
    pl.kernel,
    mesh=plsc.VectorSubcoreMesh(core_axis_name="c", subcore_axis_name="s"),
    out_type=jax.ShapeDtypeStruct((16,), jnp.float32),
    scratch_types=[
        pltpu.VMEM((_ROWS,), jnp.int32),
        pltpu.VMEM((_ROWS, _D), jnp.float32),
        pltpu.VMEM((2, _ROWS), jnp.float32),
        pltpu.VMEM((2, 16), jnp.float32),
        pltpu.VMEM((_NW, 2, 16), jnp.float32),
        pltpu.VMEM_SHARED((_NW, 2, 16), jnp.float32),
        pltpu.VMEM((16,), jnp.float32),
        pltpu.SemaphoreType.DMA,
    ],
)(_n2v_body)


@jax.jit
def kernel(embedding, source_node, context_nodes, neg_samples):
    src = source_node.astype(jnp.int32)
    payload = jnp.concatenate([context_nodes.astype(jnp.int32),
                               neg_samples.astype(jnp.int32)]).reshape(_NW, _PER_W)
    idx = jnp.concatenate(
        [jnp.broadcast_to(src[None, :], (_NW, 1)),
         payload,
         jnp.zeros((_NW, _ROWS - 1 - _PER_W), jnp.int32)], axis=1)
    out = _n2v(embedding, idx, jnp.asarray(_MASKS))
    return out[0]
